# baseline (speedup 1.0000x reference)
import functools

import jax
import jax.numpy as jnp
from jax import lax
from jax.experimental import pallas as pl
from jax.experimental.pallas import tpu as pltpu
from jax.experimental.pallas import tpu_sc as plsc

N = 10000
E = 320000
D = 128

NC = 2
NS = 16
NW = NC * NS
K = 128
G = 80
EPT = G * K
EPAD = NW * EPT - E
PADW = 240
BLK = 16
NB = 10240
ZB = NB // NS
NP = 10240
RPS = NP // NS

_mesh = plsc.VectorSubcoreMesh(core_axis_name="c", subcore_axis_name="s")


@functools.partial(
    pl.kernel,
    mesh=_mesh,
    out_type=jax.ShapeDtypeStruct((NC, 2, NB), jnp.float32),
    scratch_types=[
        pltpu.VMEM((G, K), jnp.int32),
        pltpu.VMEM((G, K), jnp.int32),
        pltpu.VMEM((K,), jnp.float32),
        pltpu.VMEM((ZB,), jnp.float32),
        pltpu.VMEM_SHARED((NB,), jnp.float32),
        pltpu.VMEM_SHARED((NB,), jnp.float32),
        pltpu.SemaphoreType.DMA,
    ],
)
def _hist_kernel(esrc_hbm, edst_hbm, out_hbm, isrc_v, idst_v, ones_v, buf_v,
                 csrc_sh, cdst_sh, sem):
    cid = lax.axis_index("c")
    sid = lax.axis_index("s")
    w = sid * NC + cid
    pltpu.make_async_copy(esrc_hbm.at[w], isrc_v, sem).start()
    pltpu.make_async_copy(edst_hbm.at[w], idst_v, sem).start()
    pltpu.make_async_copy(esrc_hbm.at[w], isrc_v, sem).wait()
    pltpu.make_async_copy(edst_hbm.at[w], idst_v, sem).wait()

    def _fill_ones(i, _):
        ones_v[pl.ds(i * 16, 16)] = jnp.ones((16,), jnp.float32)
        return 0
    lax.fori_loop(0, K // 16, _fill_ones, 0)

    def _fill_zeros(i, _):
        buf_v[pl.ds(i * 16, 16)] = jnp.zeros((16,), jnp.float32)
        return 0
    lax.fori_loop(0, ZB // 16, _fill_zeros, 0)

    pltpu.sync_copy(buf_v, csrc_sh.at[pl.ds(sid * ZB, ZB)])
    pltpu.sync_copy(buf_v, cdst_sh.at[pl.ds(sid * ZB, ZB)])
    plsc.subcore_barrier()

    WV = 8

    def _wave(wv, _):
        def _fire(g2, _):
            g = wv * WV + g2
            pltpu.make_async_copy(
                ones_v, csrc_sh.at[isrc_v.at[g]], sem).start(add=True)
            pltpu.make_async_copy(
                ones_v, cdst_sh.at[idst_v.at[g]], sem).start(add=True)
            return 0
        lax.fori_loop(0, WV, _fire, 0)

        def _drain(g2, _):
            g = wv * WV + g2
            pltpu.make_async_copy(
                ones_v, csrc_sh.at[isrc_v.at[g]], sem).wait()
            pltpu.make_async_copy(
                ones_v, cdst_sh.at[idst_v.at[g]], sem).wait()
            return 0
        lax.fori_loop(0, WV, _drain, 0)
        return 0
    lax.fori_loop(0, G // WV, _wave, 0)
    plsc.subcore_barrier()

    pltpu.sync_copy(csrc_sh.at[pl.ds(sid * ZB, ZB)], buf_v)
    pltpu.sync_copy(buf_v, out_hbm.at[cid, 0, pl.ds(sid * ZB, ZB)])
    pltpu.sync_copy(cdst_sh.at[pl.ds(sid * ZB, ZB)], buf_v)
    pltpu.sync_copy(buf_v, out_hbm.at[cid, 1, pl.ds(sid * ZB, ZB)])


@functools.partial(
    pl.kernel,
    mesh=_mesh,
    out_type=jax.ShapeDtypeStruct((NC, NP, D), jnp.float32),
    scratch_types=[
        pltpu.VMEM((BLK, K), jnp.int32),
        pltpu.VMEM((BLK, K), jnp.int32),
        pltpu.VMEM((BLK, K), jnp.int32),
        pltpu.VMEM((BLK, K), jnp.int32),
        pltpu.VMEM((K, D), jnp.float32),
        pltpu.VMEM((K, D), jnp.float32),
        pltpu.VMEM_SHARED((NP, D), jnp.float32),
        pltpu.SemaphoreType.DMA,
        pltpu.SemaphoreType.DMA,
        pltpu.SemaphoreType.DMA,
        pltpu.SemaphoreType.DMA,
    ],
)
def _agg_kernel(feat_hbm, esrc_hbm, edst_hbm, out_hbm, ias, iad, ibs, ibd,
                rows0, rows1, agg_sh, sem_a, sem_b, sem0, sem1):
    cid = lax.axis_index("c")
    sid = lax.axis_index("s")
    w = sid * NC + cid

    class _IdxPair:
        def __init__(self, s, d):
            self.s = s
            self.d = d

    idx_a = _IdxPair(ias, iad)
    idx_b = _IdxPair(ibs, ibd)

    def _stage_start(blk, ip, isem):
        pltpu.make_async_copy(
            esrc_hbm.at[w, pl.ds(blk * BLK, BLK)], ip.s, isem).start()
        pltpu.make_async_copy(
            edst_hbm.at[w, pl.ds(blk * BLK, BLK)], ip.d, isem).start()

    def _stage_wait(blk, ip, isem):
        pltpu.make_async_copy(
            esrc_hbm.at[w, pl.ds(blk * BLK, BLK)], ip.s, isem).wait()
        pltpu.make_async_copy(
            edst_hbm.at[w, pl.ds(blk * BLK, BLK)], ip.d, isem).wait()

    def _gather(ibuf, j, rbuf, rsem):
        return pltpu.make_async_copy(feat_hbm.at[ibuf.s.at[j]], rbuf, rsem)

    def _scatter(ibuf, j, rbuf):
        pltpu.sync_copy(rbuf, agg_sh.at[ibuf.d.at[j]], add=True)

    _stage_start(0, idx_a, sem_a)

    def _fill_zeros(i, _):
        rows0[i // 8, pl.ds((i % 8) * 16, 16)] = jnp.zeros((16,), jnp.float32)
        return 0
    lax.fori_loop(0, K * 8, _fill_zeros, 0)

    def _zero_agg(j, _):
        pltpu.sync_copy(rows0, agg_sh.at[pl.ds(sid * RPS + j * K, K)])
        return 0
    lax.fori_loop(0, RPS // K, _zero_agg, 0)
    plsc.subcore_barrier()

    def _block(ibuf):
        def _body(jo, _):
            j0 = jo * 2
            _gather(ibuf, j0, rows0, sem0).wait()
            _scatter(ibuf, j0, rows0)

            @pl.when(j0 + 2 < BLK)
            def _():
                _gather(ibuf, j0 + 2, rows0, sem0).start()

            j1 = j0 + 1
            _gather(ibuf, j1, rows1, sem1).wait()
            _scatter(ibuf, j1, rows1)

            @pl.when(j1 + 2 < BLK)
            def _():
                _gather(ibuf, j1 + 2, rows1, sem1).start()
            return 0
        lax.fori_loop(0, BLK // 2, _body, 0)

    bufs = [idx_a, idx_b, idx_a, idx_b, idx_a]
    sems = [sem_a, sem_b, sem_a, sem_b, sem_a]
    for blk in range(5):
        ip = bufs[blk]
        _stage_wait(blk, ip, sems[blk])
        if blk + 1 < 5:
            _stage_start(blk + 1, bufs[blk + 1], sems[blk + 1])
        _gather(ip, 0, rows0, sem0).start()
        _gather(ip, 1, rows1, sem1).start()
        _block(ip)

    plsc.subcore_barrier()

    def _readback(j, _):
        base = sid * RPS + j * K
        pltpu.sync_copy(agg_sh.at[pl.ds(base, K)], rows0)
        pltpu.sync_copy(rows0, out_hbm.at[cid, pl.ds(base, K)])
        return 0
    lax.fori_loop(0, RPS // K, _readback, 0)


def _scale_body(deg_ref, feat_ref, out_ref):
    deg = deg_ref[0, 0] + deg_ref[1, 0]
    row = lax.broadcasted_iota(jnp.int32, deg.shape, 0)
    pad0 = (pl.program_id(0) == 0) & (row < PADW)
    deg = deg - jnp.where(pad0, jnp.float32(EPAD // PADW), jnp.float32(0.0))
    scale = lax.rsqrt(jnp.maximum(deg, 1.0))
    out_ref[...] = feat_ref[...] * scale


def _final_body(aggp_ref, deg_ref, wmu_ref, wlog_ref, epsw_ref,
                bmu_ref, blog_ref, epsb_ref, out_ref, kl_ref):
    wlog = wlog_ref[...]
    wmu = wmu_ref[...]
    weight = wmu + jnp.exp(wlog) * epsw_ref[...]
    agg = aggp_ref[0] + aggp_ref[1]
    rst = jnp.dot(agg, weight, preferred_element_type=jnp.float32,
                  precision=lax.Precision.HIGHEST)
    deg = deg_ref[0, 1] + deg_ref[1, 1]
    scale = lax.rsqrt(jnp.maximum(deg, 1.0))
    blog = blog_ref[...]
    bmu = bmu_ref[...]
    bias = bmu + jnp.exp(blog) * epsb_ref[...]
    out_ref[...] = rst * scale + bias

    @pl.when(pl.program_id(0) == 0)
    def _():
        klw = jnp.sum(-wlog + (jnp.exp(2.0 * wlog) + wmu * wmu) * 0.5 - 0.5)
        klb = jnp.sum(-blog + (jnp.exp(2.0 * blog) + bmu * bmu) * 0.5 - 0.5)
        kl_ref[...] = jnp.reshape(klw + klb, (1, 1))


def kernel(feat, weight_mu, weight_logsd, bias_mu, bias_logsd, edge_index):
    ramp = jnp.arange(EPAD, dtype=jnp.int32)
    pad = jnp.concatenate([
        (ramp % PADW).reshape(1, EPAD),
        (N + ramp % (NP - N)).reshape(1, EPAD),
    ])
    edges = jnp.concatenate([edge_index.astype(jnp.int32), pad], axis=1)
    edges = edges.reshape(2, NW, G, K)
    esrc = edges[0]
    edst = edges[1]

    hist = _hist_kernel(esrc, edst)
    deg_col = hist.reshape(NC, 2, NB, 1)

    rb = 1000
    feat_scaled = pl.pallas_call(
        _scale_body,
        grid=(N // rb,),
        in_specs=[
            pl.BlockSpec((NC, 2, rb, 1), lambda i: (0, 0, i, 0)),
            pl.BlockSpec((rb, D), lambda i: (i, 0)),
        ],
        out_specs=pl.BlockSpec((rb, D), lambda i: (i, 0)),
        out_shape=jax.ShapeDtypeStruct((N, D), jnp.float32),
    )(deg_col, feat)

    aggp = _agg_kernel(feat_scaled, esrc, edst)

    eps_w = jax.random.normal(jax.random.key(42), weight_mu.shape,
                              dtype=weight_mu.dtype)
    eps_b = jax.random.normal(jax.random.key(43), bias_mu.shape,
                              dtype=bias_mu.dtype)

    rst, kl = pl.pallas_call(
        _final_body,
        grid=(N // rb,),
        in_specs=[
            pl.BlockSpec((NC, rb, D), lambda i: (0, i, 0)),
            pl.BlockSpec((NC, 2, rb, 1), lambda i: (0, 0, i, 0)),
            pl.BlockSpec((D, D), lambda i: (0, 0)),
            pl.BlockSpec((D, D), lambda i: (0, 0)),
            pl.BlockSpec((D, D), lambda i: (0, 0)),
            pl.BlockSpec((1, D), lambda i: (0, 0)),
            pl.BlockSpec((1, D), lambda i: (0, 0)),
            pl.BlockSpec((1, D), lambda i: (0, 0)),
        ],
        out_specs=[
            pl.BlockSpec((rb, D), lambda i: (i, 0)),
            pl.BlockSpec((1, 1), lambda i: (0, 0)),
        ],
        out_shape=[
            jax.ShapeDtypeStruct((N, D), jnp.float32),
            jax.ShapeDtypeStruct((1, 1), jnp.float32),
        ],
    )(aggp, deg_col, weight_mu, weight_logsd, eps_w,
      bias_mu, bias_logsd, eps_b)

    return rst, kl[0, 0]


# speedup vs baseline: 1.0793x; 1.0793x over previous
import functools

import jax
import jax.numpy as jnp
from jax import lax
from jax.experimental import pallas as pl
from jax.experimental.pallas import tpu as pltpu
from jax.experimental.pallas import tpu_sc as plsc

N = 10000
E = 320000
D = 128

NC = 2
NS = 16
NW = NC * NS
K = 128
G = 80
EPT = G * K
EPAD = NW * EPT - E
PADW = 240
BLK = 20
NB = 10240
ZB = NB // NS
NP = 10240
RPS = NP // NS

_mesh = plsc.VectorSubcoreMesh(core_axis_name="c", subcore_axis_name="s")


@functools.partial(
    pl.kernel,
    mesh=_mesh,
    out_type=jax.ShapeDtypeStruct((NC, 2, NB), jnp.float32),
    scratch_types=[
        pltpu.VMEM((G, 2, K), jnp.int32),
        pltpu.VMEM((K,), jnp.float32),
        pltpu.VMEM((ZB,), jnp.float32),
        pltpu.VMEM_SHARED((NB,), jnp.float32),
        pltpu.VMEM_SHARED((NB,), jnp.float32),
        pltpu.SemaphoreType.DMA,
    ],
)
def _hist_kernel(edges_hbm, out_hbm, idx_v, ones_v, buf_v,
                 csrc_sh, cdst_sh, sem):
    cid = lax.axis_index("c")
    sid = lax.axis_index("s")
    w = sid * NC + cid
    pltpu.sync_copy(edges_hbm.at[w], idx_v)

    def _fill_ones(i, _):
        ones_v[pl.ds(i * 16, 16)] = jnp.ones((16,), jnp.float32)
        return 0
    lax.fori_loop(0, K // 16, _fill_ones, 0)

    def _fill_zeros(i, _):
        buf_v[pl.ds(i * 16, 16)] = jnp.zeros((16,), jnp.float32)
        return 0
    lax.fori_loop(0, ZB // 16, _fill_zeros, 0)

    pltpu.sync_copy(buf_v, csrc_sh.at[pl.ds(sid * ZB, ZB)])
    pltpu.sync_copy(buf_v, cdst_sh.at[pl.ds(sid * ZB, ZB)])
    plsc.subcore_barrier()

    WV = 8

    def _wave(wv, _):
        def _fire(g2, _):
            g = wv * WV + g2
            pltpu.make_async_copy(
                ones_v, csrc_sh.at[idx_v.at[g, 0]], sem).start(add=True)
            pltpu.make_async_copy(
                ones_v, cdst_sh.at[idx_v.at[g, 1]], sem).start(add=True)
            return 0
        lax.fori_loop(0, WV, _fire, 0)

        def _drain(g2, _):
            g = wv * WV + g2
            pltpu.make_async_copy(
                ones_v, csrc_sh.at[idx_v.at[g, 0]], sem).wait()
            pltpu.make_async_copy(
                ones_v, cdst_sh.at[idx_v.at[g, 1]], sem).wait()
            return 0
        lax.fori_loop(0, WV, _drain, 0)
        return 0
    lax.fori_loop(0, G // WV, _wave, 0)
    plsc.subcore_barrier()

    pltpu.sync_copy(csrc_sh.at[pl.ds(sid * ZB, ZB)], buf_v)
    pltpu.sync_copy(buf_v, out_hbm.at[cid, 0, pl.ds(sid * ZB, ZB)])
    pltpu.sync_copy(cdst_sh.at[pl.ds(sid * ZB, ZB)], buf_v)
    pltpu.sync_copy(buf_v, out_hbm.at[cid, 1, pl.ds(sid * ZB, ZB)])


@functools.partial(
    pl.kernel,
    mesh=_mesh,
    out_type=jax.ShapeDtypeStruct((NC, NP, D), jnp.float32),
    scratch_types=[
        pltpu.VMEM((BLK, 2, K), jnp.int32),
        pltpu.VMEM((BLK, 2, K), jnp.int32),
        pltpu.VMEM((K, D), jnp.float32),
        pltpu.VMEM((K, D), jnp.float32),
        pltpu.VMEM_SHARED((NP, D), jnp.float32),
        pltpu.SemaphoreType.DMA,
        pltpu.SemaphoreType.DMA,
        pltpu.SemaphoreType.DMA,
        pltpu.SemaphoreType.DMA,
    ],
)
def _agg_kernel(feat_hbm, edges_hbm, out_hbm, idx_a, idx_b, rows0, rows1,
                agg_sh, sem_a, sem_b, sem0, sem1):
    cid = lax.axis_index("c")
    sid = lax.axis_index("s")
    w = sid * NC + cid

    def _stage_start(blk, ibuf, isem):
        pltpu.make_async_copy(
            edges_hbm.at[w, pl.ds(blk * BLK, BLK)], ibuf, isem).start()

    def _stage_wait(blk, ibuf, isem):
        pltpu.make_async_copy(
            edges_hbm.at[w, pl.ds(blk * BLK, BLK)], ibuf, isem).wait()

    def _gather(ibuf, j, rbuf, rsem):
        return pltpu.make_async_copy(feat_hbm.at[ibuf.at[j, 0]], rbuf, rsem)

    def _scatter(ibuf, j, rbuf):
        pltpu.sync_copy(rbuf, agg_sh.at[ibuf.at[j, 1]], add=True)

    _stage_start(0, idx_a, sem_a)

    def _fill_zeros(i, _):
        rows0[i // 8, pl.ds((i % 8) * 16, 16)] = jnp.zeros((16,), jnp.float32)
        return 0
    lax.fori_loop(0, K * 8, _fill_zeros, 0)

    def _zero_agg(j, _):
        pltpu.sync_copy(rows0, agg_sh.at[pl.ds(sid * RPS + j * K, K)])
        return 0
    lax.fori_loop(0, RPS // K, _zero_agg, 0)
    plsc.subcore_barrier()

    def _block(ibuf):
        def _body(jo, _):
            j0 = jo * 2
            _gather(ibuf, j0, rows0, sem0).wait()
            _scatter(ibuf, j0, rows0)

            @pl.when(j0 + 2 < BLK)
            def _():
                _gather(ibuf, j0 + 2, rows0, sem0).start()

            j1 = j0 + 1
            _gather(ibuf, j1, rows1, sem1).wait()
            _scatter(ibuf, j1, rows1)

            @pl.when(j1 + 2 < BLK)
            def _():
                _gather(ibuf, j1 + 2, rows1, sem1).start()
            return 0
        lax.fori_loop(0, BLK // 2, _body, 0)

    bufs = [idx_a, idx_b, idx_a, idx_b]
    sems = [sem_a, sem_b, sem_a, sem_b]
    for blk in range(4):
        ip = bufs[blk]
        _stage_wait(blk, ip, sems[blk])
        if blk + 1 < 4:
            _stage_start(blk + 1, bufs[blk + 1], sems[blk + 1])
        _gather(ip, 0, rows0, sem0).start()
        _gather(ip, 1, rows1, sem1).start()
        _block(ip)

    plsc.subcore_barrier()

    def _readback(j, _):
        base = sid * RPS + j * K
        pltpu.sync_copy(agg_sh.at[pl.ds(base, K)], rows0)
        pltpu.sync_copy(rows0, out_hbm.at[cid, pl.ds(base, K)])
        return 0
    lax.fori_loop(0, RPS // K, _readback, 0)


def _scale_body(deg_ref, feat_ref, out_ref):
    deg = deg_ref[0, 0] + deg_ref[1, 0]
    row = lax.broadcasted_iota(jnp.int32, deg.shape, 0)
    pad0 = (pl.program_id(0) == 0) & (row < PADW)
    deg = deg - jnp.where(pad0, jnp.float32(EPAD // PADW), jnp.float32(0.0))
    scale = lax.rsqrt(jnp.maximum(deg, 1.0))
    out_ref[...] = feat_ref[...] * scale


def _final_body(aggp_ref, deg_ref, wmu_ref, wlog_ref, epsw_ref,
                bmu_ref, blog_ref, epsb_ref, out_ref, kl_ref):
    wlog = wlog_ref[...]
    wmu = wmu_ref[...]
    weight = wmu + jnp.exp(wlog) * epsw_ref[...]
    agg = aggp_ref[0] + aggp_ref[1]
    rst = jnp.dot(agg, weight, preferred_element_type=jnp.float32,
                  precision=lax.Precision.HIGHEST)
    deg = deg_ref[0, 1] + deg_ref[1, 1]
    scale = lax.rsqrt(jnp.maximum(deg, 1.0))
    blog = blog_ref[...]
    bmu = bmu_ref[...]
    bias = bmu + jnp.exp(blog) * epsb_ref[...]
    out_ref[...] = rst * scale + bias

    @pl.when(pl.program_id(0) == 0)
    def _():
        klw = jnp.sum(-wlog + (jnp.exp(2.0 * wlog) + wmu * wmu) * 0.5 - 0.5)
        klb = jnp.sum(-blog + (jnp.exp(2.0 * blog) + bmu * bmu) * 0.5 - 0.5)
        kl_ref[...] = jnp.reshape(klw + klb, (1, 1))


def kernel(feat, weight_mu, weight_logsd, bias_mu, bias_logsd, edge_index):
    ramp = jnp.arange(EPAD, dtype=jnp.int32)
    pad = jnp.concatenate([
        (ramp % PADW).reshape(1, EPAD),
        (N + ramp % (NP - N)).reshape(1, EPAD),
    ])
    edges = jnp.concatenate([edge_index.astype(jnp.int32), pad], axis=1)
    edges = edges.reshape(2, NW, G, K).transpose(1, 2, 0, 3)

    hist = _hist_kernel(edges)
    deg_col = hist.reshape(NC, 2, NB, 1)

    rb = 2000
    feat_scaled = pl.pallas_call(
        _scale_body,
        grid=(N // rb,),
        in_specs=[
            pl.BlockSpec((NC, 2, rb, 1), lambda i: (0, 0, i, 0)),
            pl.BlockSpec((rb, D), lambda i: (i, 0)),
        ],
        out_specs=pl.BlockSpec((rb, D), lambda i: (i, 0)),
        out_shape=jax.ShapeDtypeStruct((N, D), jnp.float32),
    )(deg_col, feat)

    aggp = _agg_kernel(feat_scaled, edges)

    eps_w = jax.random.normal(jax.random.key(42), weight_mu.shape,
                              dtype=weight_mu.dtype)
    eps_b = jax.random.normal(jax.random.key(43), bias_mu.shape,
                              dtype=bias_mu.dtype)

    rst, kl = pl.pallas_call(
        _final_body,
        grid=(N // rb,),
        in_specs=[
            pl.BlockSpec((NC, rb, D), lambda i: (0, i, 0)),
            pl.BlockSpec((NC, 2, rb, 1), lambda i: (0, 0, i, 0)),
            pl.BlockSpec((D, D), lambda i: (0, 0)),
            pl.BlockSpec((D, D), lambda i: (0, 0)),
            pl.BlockSpec((D, D), lambda i: (0, 0)),
            pl.BlockSpec((1, D), lambda i: (0, 0)),
            pl.BlockSpec((1, D), lambda i: (0, 0)),
            pl.BlockSpec((1, D), lambda i: (0, 0)),
        ],
        out_specs=[
            pl.BlockSpec((rb, D), lambda i: (i, 0)),
            pl.BlockSpec((1, 1), lambda i: (0, 0)),
        ],
        out_shape=[
            jax.ShapeDtypeStruct((N, D), jnp.float32),
            jax.ShapeDtypeStruct((1, 1), jnp.float32),
        ],
    )(aggp, deg_col, weight_mu, weight_logsd, eps_w,
      bias_mu, bias_logsd, eps_b)

    return rst, kl[0, 0]


# speedup vs baseline: 1.2166x; 1.1272x over previous
import functools

import jax
import jax.numpy as jnp
from jax import lax
from jax.experimental import pallas as pl
from jax.experimental.pallas import tpu as pltpu
from jax.experimental.pallas import tpu_sc as plsc

N = 10000
E = 320000
D = 128

NC = 2
NS = 16
NW = NC * NS
K = 128
G = 80
EPT = G * K
EPAD = NW * EPT - E
PADW = 240
BLK = 20
NB = 10240
ZB = NB // NS
NP = 10240
RPS = NP // NS

_mesh = plsc.VectorSubcoreMesh(core_axis_name="c", subcore_axis_name="s")


@functools.partial(
    pl.kernel,
    mesh=_mesh,
    out_type=jax.ShapeDtypeStruct((NC, 2, NB), jnp.float32),
    scratch_types=[
        pltpu.VMEM((G, 2, K), jnp.int32),
        pltpu.VMEM((K,), jnp.float32),
        pltpu.VMEM((ZB,), jnp.float32),
        pltpu.VMEM_SHARED((NB,), jnp.float32),
        pltpu.VMEM_SHARED((NB,), jnp.float32),
        pltpu.SemaphoreType.DMA,
    ],
)
def _hist_kernel(edges_hbm, out_hbm, idx_v, ones_v, buf_v,
                 csrc_sh, cdst_sh, sem):
    cid = lax.axis_index("c")
    sid = lax.axis_index("s")
    w = sid * NC + cid
    pltpu.sync_copy(edges_hbm.at[w], idx_v)

    def _fill_ones(i, _):
        ones_v[pl.ds(i * 16, 16)] = jnp.ones((16,), jnp.float32)
        return 0
    lax.fori_loop(0, K // 16, _fill_ones, 0)

    def _fill_zeros(i, _):
        buf_v[pl.ds(i * 16, 16)] = jnp.zeros((16,), jnp.float32)
        return 0
    lax.fori_loop(0, ZB // 16, _fill_zeros, 0)

    pltpu.sync_copy(buf_v, csrc_sh.at[pl.ds(sid * ZB, ZB)])
    pltpu.sync_copy(buf_v, cdst_sh.at[pl.ds(sid * ZB, ZB)])
    plsc.subcore_barrier()

    WV = 8

    def _wave(wv, _):
        def _fire(g2, _):
            g = wv * WV + g2
            pltpu.make_async_copy(
                ones_v, csrc_sh.at[idx_v.at[g, 0]], sem).start(add=True)
            pltpu.make_async_copy(
                ones_v, cdst_sh.at[idx_v.at[g, 1]], sem).start(add=True)
            return 0
        lax.fori_loop(0, WV, _fire, 0)

        def _drain(g2, _):
            g = wv * WV + g2
            pltpu.make_async_copy(
                ones_v, csrc_sh.at[idx_v.at[g, 0]], sem).wait()
            pltpu.make_async_copy(
                ones_v, cdst_sh.at[idx_v.at[g, 1]], sem).wait()
            return 0
        lax.fori_loop(0, WV, _drain, 0)
        return 0
    lax.fori_loop(0, G // WV, _wave, 0)
    plsc.subcore_barrier()

    pltpu.sync_copy(csrc_sh.at[pl.ds(sid * ZB, ZB)], buf_v)
    pltpu.sync_copy(buf_v, out_hbm.at[cid, 0, pl.ds(sid * ZB, ZB)])
    pltpu.sync_copy(cdst_sh.at[pl.ds(sid * ZB, ZB)], buf_v)
    pltpu.sync_copy(buf_v, out_hbm.at[cid, 1, pl.ds(sid * ZB, ZB)])


@functools.partial(
    pl.kernel,
    mesh=_mesh,
    out_type=jax.ShapeDtypeStruct((NC, NP, D), jnp.float32),
    scratch_types=[
        pltpu.VMEM((BLK, 2, K), jnp.int32),
        pltpu.VMEM((BLK, 2, K), jnp.int32),
        pltpu.VMEM((K, D), jnp.float32),
        pltpu.VMEM((K, D), jnp.float32),
        pltpu.VMEM_SHARED((NP, D), jnp.float32),
        pltpu.SemaphoreType.DMA,
        pltpu.SemaphoreType.DMA,
        pltpu.SemaphoreType.DMA,
        pltpu.SemaphoreType.DMA,
    ],
)
def _agg_kernel(feat_hbm, edges_hbm, out_hbm, idx_a, idx_b, rows0, rows1,
                agg_sh, sem_a, sem_b, sem0, sem1):
    cid = lax.axis_index("c")
    sid = lax.axis_index("s")
    w = sid * NC + cid

    def _stage_start(blk, ibuf, isem):
        pltpu.make_async_copy(
            edges_hbm.at[w, pl.ds(blk * BLK, BLK)], ibuf, isem).start()

    def _stage_wait(blk, ibuf, isem):
        pltpu.make_async_copy(
            edges_hbm.at[w, pl.ds(blk * BLK, BLK)], ibuf, isem).wait()

    def _gather(ibuf, j, rbuf, rsem):
        return pltpu.make_async_copy(feat_hbm.at[ibuf.at[j, 0]], rbuf, rsem)

    def _scatter(ibuf, j, rbuf):
        pltpu.sync_copy(rbuf, agg_sh.at[ibuf.at[j, 1]], add=True)

    _stage_start(0, idx_a, sem_a)

    def _fill_zeros(i, _):
        rows0[i // 8, pl.ds((i % 8) * 16, 16)] = jnp.zeros((16,), jnp.float32)
        return 0
    lax.fori_loop(0, K * 8, _fill_zeros, 0)

    def _zero_agg(j, _):
        pltpu.sync_copy(rows0, agg_sh.at[pl.ds(sid * RPS + j * K, K)])
        return 0
    lax.fori_loop(0, RPS // K, _zero_agg, 0)
    plsc.subcore_barrier()

    def _block(ibuf):
        def _body(jo, _):
            j0 = jo * 2
            _gather(ibuf, j0, rows0, sem0).wait()
            _scatter(ibuf, j0, rows0)

            @pl.when(j0 + 2 < BLK)
            def _():
                _gather(ibuf, j0 + 2, rows0, sem0).start()

            j1 = j0 + 1
            _gather(ibuf, j1, rows1, sem1).wait()
            _scatter(ibuf, j1, rows1)

            @pl.when(j1 + 2 < BLK)
            def _():
                _gather(ibuf, j1 + 2, rows1, sem1).start()
            return 0
        lax.fori_loop(0, BLK // 2, _body, 0)

    bufs = [idx_a, idx_b, idx_a, idx_b]
    sems = [sem_a, sem_b, sem_a, sem_b]
    for blk in range(4):
        ip = bufs[blk]
        _stage_wait(blk, ip, sems[blk])
        if blk + 1 < 4:
            _stage_start(blk + 1, bufs[blk + 1], sems[blk + 1])
        _gather(ip, 0, rows0, sem0).start()
        _gather(ip, 1, rows1, sem1).start()
        _block(ip)

    plsc.subcore_barrier()

    def _readback(j, _):
        base = sid * RPS + j * K
        pltpu.sync_copy(agg_sh.at[pl.ds(base, K)], rows0)
        pltpu.sync_copy(rows0, out_hbm.at[cid, pl.ds(base, K)])
        return 0
    lax.fori_loop(0, RPS // K, _readback, 0)


def _scale_body(deg_ref, feat_ref, out_ref):
    rb = feat_ref.shape[0]
    base = pl.program_id(0) * rb
    deg = (deg_ref[0, 0, pl.ds(base, rb)] +
           deg_ref[1, 0, pl.ds(base, rb)])
    row = lax.broadcasted_iota(jnp.int32, deg.shape, 0)
    pad0 = (pl.program_id(0) == 0) & (row < PADW)
    deg = deg - jnp.where(pad0, jnp.float32(EPAD // PADW), jnp.float32(0.0))
    scale = lax.rsqrt(jnp.maximum(deg, 1.0))
    out_ref[...] = feat_ref[...] * scale[:, None]


def _final_body(aggp_ref, deg_ref, wmu_ref, wlog_ref, epsw_ref,
                bmu_ref, blog_ref, epsb_ref, out_ref, kl_ref):
    wlog = wlog_ref[...]
    wmu = wmu_ref[...]
    weight = wmu + jnp.exp(wlog) * epsw_ref[...]
    agg = aggp_ref[0] + aggp_ref[1]
    rst = jnp.dot(agg, weight, preferred_element_type=jnp.float32)
    rb = out_ref.shape[0]
    base = pl.program_id(0) * rb
    deg = (deg_ref[0, 1, pl.ds(base, rb)] +
           deg_ref[1, 1, pl.ds(base, rb)])
    scale = lax.rsqrt(jnp.maximum(deg, 1.0))[:, None]
    blog = blog_ref[...]
    bmu = bmu_ref[...]
    bias = bmu + jnp.exp(blog) * epsb_ref[...]
    out_ref[...] = rst * scale + bias

    @pl.when(pl.program_id(0) == 0)
    def _():
        klw = jnp.sum(-wlog + (jnp.exp(2.0 * wlog) + wmu * wmu) * 0.5 - 0.5)
        klb = jnp.sum(-blog + (jnp.exp(2.0 * blog) + bmu * bmu) * 0.5 - 0.5)
        kl_ref[...] = jnp.reshape(klw + klb, (1, 1))


def kernel(feat, weight_mu, weight_logsd, bias_mu, bias_logsd, edge_index):
    ramp = jnp.arange(EPAD, dtype=jnp.int32)
    pad = jnp.concatenate([
        (ramp % PADW).reshape(1, EPAD),
        (N + ramp % (NP - N)).reshape(1, EPAD),
    ])
    edges = jnp.concatenate([edge_index.astype(jnp.int32), pad], axis=1)
    edges = edges.reshape(2, NW, G, K).transpose(1, 2, 0, 3)

    deg_col = _hist_kernel(edges)

    rb = 2048
    feat_scaled = pl.pallas_call(
        _scale_body,
        grid=((N + rb - 1) // rb,),
        in_specs=[
            pl.BlockSpec((NC, 2, NB), lambda i: (0, 0, 0)),
            pl.BlockSpec((rb, D), lambda i: (i, 0)),
        ],
        out_specs=pl.BlockSpec((rb, D), lambda i: (i, 0)),
        out_shape=jax.ShapeDtypeStruct((N, D), jnp.float32),
    )(deg_col, feat)

    aggp = _agg_kernel(feat_scaled, edges)

    eps_w = jax.random.normal(jax.random.key(42), weight_mu.shape,
                              dtype=weight_mu.dtype)
    eps_b = jax.random.normal(jax.random.key(43), bias_mu.shape,
                              dtype=bias_mu.dtype)

    rst, kl = pl.pallas_call(
        _final_body,
        grid=((N + rb - 1) // rb,),
        in_specs=[
            pl.BlockSpec((NC, rb, D), lambda i: (0, i, 0)),
            pl.BlockSpec((NC, 2, NB), lambda i: (0, 0, 0)),
            pl.BlockSpec((D, D), lambda i: (0, 0)),
            pl.BlockSpec((D, D), lambda i: (0, 0)),
            pl.BlockSpec((D, D), lambda i: (0, 0)),
            pl.BlockSpec((1, D), lambda i: (0, 0)),
            pl.BlockSpec((1, D), lambda i: (0, 0)),
            pl.BlockSpec((1, D), lambda i: (0, 0)),
        ],
        out_specs=[
            pl.BlockSpec((rb, D), lambda i: (i, 0)),
            pl.BlockSpec((1, 1), lambda i: (0, 0)),
        ],
        out_shape=[
            jax.ShapeDtypeStruct((N, D), jnp.float32),
            jax.ShapeDtypeStruct((1, 1), jnp.float32),
        ],
    )(aggp, deg_col, weight_mu, weight_logsd, eps_w,
      bias_mu, bias_logsd, eps_b)

    return rst, kl[0, 0]
